# Initial kernel scaffold; baseline (speedup 1.0000x reference)
import jax
import jax.numpy as jnp
from jax.experimental import pallas as pl


def kernel(x, W, b):
    raise NotImplementedError("write your pallas kernel here")


# speedup vs baseline: 6.1140x; 6.1140x over previous
import jax
import jax.numpy as jnp
from jax.experimental import pallas as pl
from jax.experimental.pallas import tpu as pltpu

NUM_EXPERTS = 64
TOP_K = 8
D_MODEL = 4096
N_TOKENS = 16384
BN = 1024


def _router_body(x_ref, w_ref, b_ref, out_ref, loss_ref, acc_ref):
    i = pl.program_id(0)
    nsteps = pl.num_programs(0)

    logits = jax.lax.dot_general(
        x_ref[...], w_ref[...],
        dimension_numbers=(((1,), (1,)), ((), ())),
        preferred_element_type=jnp.float32,
    ) + b_ref[...]

    iota = jax.lax.broadcasted_iota(jnp.int32, logits.shape, 1)
    run = logits
    sel = jnp.zeros(logits.shape, jnp.bool_)
    m0 = None
    for k in range(TOP_K):
        m = jnp.max(run, axis=-1, keepdims=True)
        if k == 0:
            m0 = m
        ism = run == m
        minidx = jnp.min(jnp.where(ism, iota, NUM_EXPERTS), axis=-1,
                         keepdims=True)
        first = iota == minidx
        sel = jnp.logical_or(sel, first)
        run = jnp.where(first, -jnp.inf, run)

    e = jnp.where(sel, jnp.exp(logits - m0), 0.0)
    denom = jnp.sum(e, axis=-1, keepdims=True)
    r = e / denom
    out_ref[...] = r

    csum = jnp.sum(r, axis=0, keepdims=True)

    @pl.when(i == 0)
    def _():
        acc_ref[...] = csum

    @pl.when(i > 0)
    def _():
        acc_ref[...] = acc_ref[...] + csum

    @pl.when(i == nsteps - 1)
    def _():
        cs = acc_ref[...]
        total = jnp.sum(cs)
        usage = cs / total
        loss_ref[...] = jnp.sum((usage - 1.0 / NUM_EXPERTS) ** 2,
                                keepdims=True).reshape(1, 1)


def kernel(x, W, b):
    n = x.shape[0]
    grid = (n // BN,)
    routing, loss = pl.pallas_call(
        _router_body,
        grid=grid,
        in_specs=[
            pl.BlockSpec((BN, D_MODEL), lambda i: (i, 0)),
            pl.BlockSpec((NUM_EXPERTS, D_MODEL), lambda i: (0, 0)),
            pl.BlockSpec((1, NUM_EXPERTS), lambda i: (0, 0)),
        ],
        out_specs=[
            pl.BlockSpec((BN, NUM_EXPERTS), lambda i: (i, 0)),
            pl.BlockSpec((1, 1), lambda i: (0, 0)),
        ],
        out_shape=[
            jax.ShapeDtypeStruct((n, NUM_EXPERTS), jnp.float32),
            jax.ShapeDtypeStruct((1, 1), jnp.float32),
        ],
        scratch_shapes=[pltpu.VMEM((1, NUM_EXPERTS), jnp.float32)],
    )(x, W, b.reshape(1, NUM_EXPERTS))
    return routing, loss[0, 0]


# speedup vs baseline: 7.0354x; 1.1507x over previous
import jax
import jax.numpy as jnp
from jax.experimental import pallas as pl
from jax.experimental.pallas import tpu as pltpu

NUM_EXPERTS = 64
TOP_K = 8
D_MODEL = 4096
N_TOKENS = 16384
BN = 1024

def _router_body(x_ref, w_ref, b_ref, out_ref, loss_ref, acc_ref):
    i = pl.program_id(0)
    nsteps = pl.num_programs(0)

    lt = jax.lax.dot_general(
        w_ref[...], x_ref[...],
        dimension_numbers=(((1,), (1,)), ((), ())),
        preferred_element_type=jnp.float32,
    ) + b_ref[...]

    m0 = jnp.max(lt, axis=0, keepdims=True)

    u = jax.lax.bitcast_convert_type(lt, jnp.uint32)
    k = jnp.where(u >= jnp.uint32(0x80000000), ~u, u | jnp.uint32(0x80000000))
    idx_rev = jax.lax.broadcasted_iota(jnp.uint32, lt.shape, 0)
    k = (k & jnp.uint32(0xFFFFFFC0)) | (jnp.uint32(63) - idx_rev)
    ak = jax.lax.bitcast_convert_type(k ^ jnp.uint32(0x80000000), jnp.int32)

    run = ak
    m8 = None
    for _ in range(TOP_K):
        m8 = jnp.max(run, axis=0, keepdims=True)
        run = jnp.where(run == m8, jnp.int32(-2147483648), run)

    sel = ak >= m8
    e = jnp.where(sel, jnp.exp(lt - m0), 0.0)
    denom = jnp.sum(e, axis=0, keepdims=True)
    r = e * (1.0 / denom)
    out_ref[...] = r.T

    @pl.when(i == 0)
    def _():
        acc_ref[...] = r

    @pl.when(i > 0)
    def _():
        acc_ref[...] = acc_ref[...] + r

    @pl.when(i == nsteps - 1)
    def _():
        cs = jnp.sum(acc_ref[...], axis=1, keepdims=True)
        total = jnp.sum(cs)
        usage = cs / total
        loss_ref[...] = jnp.sum((usage - 1.0 / NUM_EXPERTS) ** 2,
                                keepdims=True).reshape(1, 1)


def kernel(x, W, b):
    n = x.shape[0]
    grid = (n // BN,)
    routing, loss = pl.pallas_call(
        _router_body,
        grid=grid,
        in_specs=[
            pl.BlockSpec((BN, D_MODEL), lambda i: (i, 0)),
            pl.BlockSpec((NUM_EXPERTS, D_MODEL), lambda i: (0, 0)),
            pl.BlockSpec((NUM_EXPERTS, 1), lambda i: (0, 0)),
        ],
        out_specs=[
            pl.BlockSpec((BN, NUM_EXPERTS), lambda i: (i, 0)),
            pl.BlockSpec((1, 1), lambda i: (0, 0)),
        ],
        out_shape=[
            jax.ShapeDtypeStruct((n, NUM_EXPERTS), jnp.float32),
            jax.ShapeDtypeStruct((1, 1), jnp.float32),
        ],
        scratch_shapes=[pltpu.VMEM((NUM_EXPERTS, BN), jnp.float32)],
    )(x, W, b.reshape(NUM_EXPERTS, 1))
    return routing, loss[0, 0]
